# Initial kernel scaffold; baseline (speedup 1.0000x reference)
import jax
import jax.numpy as jnp
from jax.experimental import pallas as pl


def kernel(x, embedding, W1, b1, W2, b2):
    raise NotImplementedError("write your pallas kernel here")


# speedup vs baseline: 13.3877x; 13.3877x over previous
import functools

import jax
import jax.numpy as jnp
from jax import lax
from jax.experimental import pallas as pl
from jax.experimental.pallas import tpu as pltpu
from jax.experimental.pallas import tpu_sc as plsc

NC = 2
NS = 16
NW = NC * NS
LANES = 16


def _make_pool(vocab, embed, batch, seq_chunks, chunk):
  rows_per_w = batch // NW
  nreg = embed // LANES
  mesh = plsc.VectorSubcoreMesh(
      core_axis_name="c", subcore_axis_name="s",
      num_cores=NC, num_subcores=NS)

  def body(x_hbm, emb_hbm, out_hbm, idx_v, buf_v, acc_v, sem0, sem1):
    wid = lax.axis_index("s") * NC + lax.axis_index("c")
    base = wid * rows_per_w
    pltpu.sync_copy(x_hbm.at[pl.ds(base, rows_per_w)], idx_v)

    sems = (sem0, sem1)

    def fire(b, p):
      for j in range(seq_chunks):
        pltpu.async_copy(emb_hbm.at[idx_v.at[b, j]], buf_v.at[p, j], sems[p])

    def wait(p):
      for j in range(seq_chunks):
        pltpu.make_async_copy(
            emb_hbm.at[idx_v.at[0, j]], buf_v.at[p, j], sems[p]).wait()

    fire(0, 0)
    fire(1, 1)

    def accum(p, b):
      def tok(j):
        def f(t, acc):
          return tuple(
              acc[k] + buf_v[p, j, t, pl.ds(LANES * k, LANES)]
              for k in range(nreg))
        return f
      acc = tuple(jnp.zeros((LANES,), jnp.float32) for _ in range(nreg))
      for j in range(seq_chunks):
        acc = lax.fori_loop(0, chunk, tok(j), acc)
      for k in range(nreg):
        acc_v[b, pl.ds(LANES * k, LANES)] = acc[k]

    def step(i, carry):
      for p in range(2):
        b = 2 * i + p
        wait(p)
        accum(p, b)

        @pl.when(b + 2 < rows_per_w)
        def _():
          fire(b + 2, p)
      return carry

    lax.fori_loop(0, rows_per_w // 2, step, 0)
    pltpu.sync_copy(acc_v, out_hbm.at[pl.ds(base, rows_per_w)])

  return pl.kernel(
      body,
      out_type=jax.ShapeDtypeStruct((batch, embed), jnp.float32),
      mesh=mesh,
      scratch_types=[
          pltpu.VMEM((rows_per_w, seq_chunks, chunk), jnp.int32),
          pltpu.VMEM((2, seq_chunks, chunk, embed), jnp.float32),
          pltpu.VMEM((rows_per_w, embed), jnp.float32),
          pltpu.SemaphoreType.DMA,
          pltpu.SemaphoreType.DMA,
      ],
  )


def _mlp_body(inv_l, p_ref, w1_ref, b1_ref, w2_ref, b2_ref, o_ref):
  pooled = p_ref[:] * inv_l
  h = jnp.maximum(
      jnp.dot(pooled, w1_ref[:], preferred_element_type=jnp.float32)
      + b1_ref[:], 0.0)
  o_ref[:] = (
      jnp.dot(h, w2_ref[:], preferred_element_type=jnp.float32) + b2_ref[:])


def kernel(x, embedding, W1, b1, W2, b2):
  batch, seq = x.shape
  vocab, embed = embedding.shape
  hidden = W1.shape[1]
  ncls = W2.shape[1]
  chunk = 100
  seq_chunks = seq // chunk

  xr = x.astype(jnp.int32).reshape(batch, seq_chunks, chunk)
  pool = _make_pool(vocab, embed, batch, seq_chunks, chunk)
  pooled_sum = pool(xr, embedding)

  ncls_pad = 128
  W2p = jnp.pad(W2, ((0, 0), (0, ncls_pad - ncls)))
  b2p = jnp.pad(b2, (0, ncls_pad - ncls)).reshape(1, ncls_pad)

  mlp = pl.pallas_call(
      functools.partial(_mlp_body, 1.0 / seq),
      out_shape=jax.ShapeDtypeStruct((batch, ncls_pad), jnp.float32),
  )
  out = mlp(pooled_sum, W1, b1.reshape(1, hidden), W2p, b2p)
  return out[:, :ncls]


# speedup vs baseline: 16.1488x; 1.2062x over previous
import functools

import jax
import jax.numpy as jnp
from jax import lax
from jax.experimental import pallas as pl
from jax.experimental.pallas import tpu as pltpu
from jax.experimental.pallas import tpu_sc as plsc

NC = 2
NS = 16
NW = NC * NS
LANES = 16


NBUF = 3
UNROLL = 4


def _make_pool(vocab, embed, batch, seq_chunks, chunk):
  rows_per_w = batch // NW
  nreg = embed // LANES
  mesh = plsc.VectorSubcoreMesh(
      core_axis_name="c", subcore_axis_name="s",
      num_cores=NC, num_subcores=NS)

  def body(x_hbm, emb_hbm, out_hbm, idx_v, buf_v, acc_v, *sems):
    wid = lax.axis_index("s") * NC + lax.axis_index("c")
    base = wid * rows_per_w
    pltpu.sync_copy(x_hbm.at[pl.ds(base, rows_per_w)], idx_v)

    def fire(b, p):
      for j in range(seq_chunks):
        pltpu.async_copy(emb_hbm.at[idx_v.at[b, j]], buf_v.at[p, j], sems[p])

    def wait(p):
      for j in range(seq_chunks):
        pltpu.make_async_copy(
            emb_hbm.at[idx_v.at[0, j]], buf_v.at[p, j], sems[p]).wait()

    for p in range(NBUF):
      fire(p, p)

    def accum(p, b):
      def tok(j):
        def f(t, acc):
          for u in range(UNROLL):
            acc = tuple(
                acc[k] + buf_v[p, j, UNROLL * t + u, pl.ds(LANES * k, LANES)]
                for k in range(nreg))
          return acc
        return f
      acc = tuple(jnp.zeros((LANES,), jnp.float32) for _ in range(nreg))
      for j in range(seq_chunks):
        acc = lax.fori_loop(0, chunk // UNROLL, tok(j), acc)
      for k in range(nreg):
        acc_v[b, pl.ds(LANES * k, LANES)] = acc[k]

    main_iters = rows_per_w // NBUF

    def step(i, carry):
      for p in range(NBUF):
        b = NBUF * i + p
        wait(p)
        accum(p, b)

        @pl.when(b + NBUF < rows_per_w)
        def _():
          fire(b + NBUF, p)
      return carry

    lax.fori_loop(0, main_iters, step, 0)
    for b in range(NBUF * main_iters, rows_per_w):
      p = b % NBUF
      wait(p)
      accum(p, b)
    pltpu.sync_copy(acc_v, out_hbm.at[pl.ds(base, rows_per_w)])

  return pl.kernel(
      body,
      out_type=jax.ShapeDtypeStruct((batch, embed), jnp.float32),
      mesh=mesh,
      scratch_types=[
          pltpu.VMEM((rows_per_w, seq_chunks, chunk), jnp.int32),
          pltpu.VMEM((NBUF, seq_chunks, chunk, embed), jnp.float32),
          pltpu.VMEM((rows_per_w, embed), jnp.float32),
      ] + [pltpu.SemaphoreType.DMA] * NBUF,
  )


def _mlp_body(inv_l, p_ref, w1_ref, b1_ref, w2_ref, b2_ref, o_ref):
  pooled = p_ref[:] * inv_l
  h = jnp.maximum(
      jnp.dot(pooled, w1_ref[:], preferred_element_type=jnp.float32)
      + b1_ref[:], 0.0)
  o_ref[:] = (
      jnp.dot(h, w2_ref[:], preferred_element_type=jnp.float32) + b2_ref[:])


def kernel(x, embedding, W1, b1, W2, b2):
  batch, seq = x.shape
  vocab, embed = embedding.shape
  hidden = W1.shape[1]
  ncls = W2.shape[1]
  chunk = 100
  seq_chunks = seq // chunk

  xr = x.astype(jnp.int32).reshape(batch, seq_chunks, chunk)
  pool = _make_pool(vocab, embed, batch, seq_chunks, chunk)
  pooled_sum = pool(xr, embedding)

  mlp = pl.pallas_call(
      functools.partial(_mlp_body, 1.0 / seq),
      out_shape=jax.ShapeDtypeStruct((batch, ncls), jnp.float32),
  )
  return mlp(pooled_sum, W1, b1.reshape(1, hidden), W2, b2.reshape(1, ncls))
